# baseline (speedup 1.0000x reference)
import functools

import jax
import jax.numpy as jnp
from jax import lax
from jax.experimental import pallas as pl
from jax.experimental.pallas import tpu as pltpu
from jax.experimental.pallas import tpu_sc as plsc

N = 10000
E = 320000
D_IN = 128
D_OUT = 64
B = 100
NPG = 100
STRIDE = 104
NR = B * STRIDE
NC = 2
NSUB = 16
NT = NC * NSUB
K = 128
CH = 80
EPT = CH * K
EP = NT * EPT
PAD = EP - E
HL = 10496
HSTR = HL // NSUB
ABUF = 10496
ASTR = ABUF // NSUB
JC = 1
SC_ROWS = JC * K
NSTEP = CH // JC
HROWS = 640
DSC = 512
DCH = EPT // DSC
AZ = 41
XP = 10240
BLK = 1024

def _deg_body(src_hbm, dst_hbm, od_hbm, id_hbm,
              idx_s, idx_d, ones_v, zv, hist_o, hist_i):
    cid = lax.axis_index("c")
    sid = lax.axis_index("s")
    wid = cid * NSUB + sid
    pltpu.sync_copy(src_hbm.at[wid], idx_s)
    pltpu.sync_copy(dst_hbm.at[wid], idx_d)

    def fill_ones(i, _):
        ones_v[0, pl.ds(i * 16, 16)] = jnp.full((16,), 1.0, jnp.float32)
        return 0

    lax.fori_loop(0, DSC // 16, fill_ones, 0)

    def fill_zero(i, _):
        zv[0, pl.ds(i * 16, 16)] = jnp.zeros((16,), jnp.float32)
        return 0

    lax.fori_loop(0, HSTR // 16, fill_zero, 0)
    pltpu.sync_copy(zv, hist_o.at[pl.ds(0, 1), pl.ds(sid * HSTR, HSTR)])
    pltpu.sync_copy(zv, hist_i.at[pl.ds(0, 1), pl.ds(sid * HSTR, HSTR)])
    plsc.subcore_barrier()

    def chunk(c, _):
        pltpu.sync_copy(ones_v, hist_o.at[idx_s.at[pl.ds(c, 1)]], add=True)
        pltpu.sync_copy(ones_v, hist_i.at[idx_d.at[pl.ds(c, 1)]], add=True)
        return 0

    lax.fori_loop(0, DCH, chunk, 0)
    plsc.subcore_barrier()
    pltpu.sync_copy(hist_o.at[pl.ds(0, 1), pl.ds(sid * HSTR, HSTR)], zv)
    pltpu.sync_copy(zv, od_hbm.at[cid, pl.ds(0, 1), pl.ds(sid * HSTR, HSTR)])
    pltpu.sync_copy(hist_i.at[pl.ds(0, 1), pl.ds(sid * HSTR, HSTR)], zv)
    pltpu.sync_copy(zv, id_hbm.at[cid, pl.ds(0, 1), pl.ds(sid * HSTR, HSTR)])


def _agg_body(h_hbm, h4_hbm, src_hbm, dst_hbm, out_hbm,
              idx_s, idx_d, rows, zv, h_sp, agg,
              sem_ga, sem_gb, sem_sa, sem_sb):
    cid = lax.axis_index("c")
    sid = lax.axis_index("s")
    wid = cid * NSUB + sid
    pltpu.sync_copy(src_hbm.at[wid], idx_s)
    pltpu.sync_copy(dst_hbm.at[wid], idx_d)

    def fill_zero(i, _):
        zv[i // 4, pl.ds((i % 4) * 16, 16)] = jnp.zeros((16,), jnp.float32)
        return 0

    lax.fori_loop(0, AZ * 4, fill_zero, 0)
    for q in range(ASTR // AZ):
        pltpu.sync_copy(zv, agg.at[0, pl.ds(sid * ASTR + q * AZ, AZ)])
    for q in range(HROWS // SC_ROWS):
        base = sid * HROWS + q * SC_ROWS
        pltpu.sync_copy(h_hbm.at[0, pl.ds(base, SC_ROWS)], rows.at[0, 0])
        pltpu.sync_copy(rows.at[0, 0], h_sp.at[0, pl.ds(base, SC_ROWS)])
    plsc.subcore_barrier()

    g_sems = (sem_ga, sem_gb)
    s_sems = (sem_sa, sem_sb)

    def wait_buf(b, sem):
        pltpu.make_async_copy(h4_hbm.at[pl.ds(0, 1)], rows.at[b], sem).wait()

    def issue_gather(s, b):
        pltpu.async_copy(h_sp.at[idx_s.at[pl.ds(s, 1)]], rows.at[b],
                         g_sems[b])

    def issue_scatter(s, b):
        pltpu.async_copy(rows.at[b], agg.at[idx_d.at[pl.ds(s, 1)]],
                         s_sems[b], add=True)

    issue_gather(0, 0)
    issue_gather(1, 1)
    for s in range(NSTEP):
        b = s % 2
        wait_buf(b, g_sems[b])
        issue_scatter(s, b)
        if s + 2 < NSTEP:
            wait_buf(b, s_sems[b])
            issue_gather(s + 2, b)
    wait_buf(0, s_sems[0])
    wait_buf(1, s_sems[1])
    plsc.subcore_barrier()
    for q in range(ASTR // AZ):
        pltpu.sync_copy(agg.at[0, pl.ds(sid * ASTR + q * AZ, AZ)], zv)
        pltpu.sync_copy(zv, out_hbm.at[cid, pl.ds(sid * ASTR + q * AZ, AZ)])


@functools.cache
def _sc_kernels():
    mesh = plsc.VectorSubcoreMesh(
        core_axis_name="c", subcore_axis_name="s",
        num_cores=NC, num_subcores=NSUB,
    )
    params = pltpu.CompilerParams(use_tc_tiling_on_sc=False)
    deg_kernel = pl.kernel(
        _deg_body,
        out_type=[
            jax.ShapeDtypeStruct((NC, 1, HL), jnp.float32),
            jax.ShapeDtypeStruct((NC, 1, HL), jnp.float32),
        ],
        mesh=mesh,
        compiler_params=params,
        scratch_types=[
            pltpu.VMEM((DCH, DSC), jnp.int32),
            pltpu.VMEM((DCH, DSC), jnp.int32),
            pltpu.VMEM((1, DSC), jnp.float32),
            pltpu.VMEM((1, HSTR), jnp.float32),
            pltpu.VMEM_SHARED((1, HL), jnp.float32),
            pltpu.VMEM_SHARED((1, HL), jnp.float32),
        ],
    )
    agg_kernel = pl.kernel(
        _agg_body,
        out_type=jax.ShapeDtypeStruct((NC, ABUF, D_OUT), jnp.float32),
        mesh=mesh,
        compiler_params=params,
        scratch_types=[
            pltpu.VMEM((NSTEP, SC_ROWS), jnp.int32),
            pltpu.VMEM((NSTEP, SC_ROWS), jnp.int32),
            pltpu.VMEM((2, 1, SC_ROWS, D_OUT), jnp.float32),
            pltpu.VMEM((AZ, D_OUT), jnp.float32),
            pltpu.VMEM_SHARED((1, XP, D_OUT), jnp.float32),
            pltpu.VMEM_SHARED((1, ABUF, D_OUT), jnp.float32),
            pltpu.SemaphoreType.DMA,
            pltpu.SemaphoreType.DMA,
            pltpu.SemaphoreType.DMA,
            pltpu.SemaphoreType.DMA,
        ],
    )
    return deg_kernel, agg_kernel


def _mm_body(x_ref, w_ref, od_ref, h_ref):
    od = od_ref[0] + od_ref[1]
    pid = pl.program_id(0)
    iota = lax.broadcasted_iota(jnp.int32, (BLK, 1), 0)
    od = jnp.where((pid == 0) & (iota == 0), od - float(PAD), od)
    norm = lax.rsqrt(jnp.maximum(od, 1.0))
    h_ref[...] = jnp.dot(x_ref[...] * norm, w_ref[...],
                         preferred_element_type=jnp.float32)


_matmul_call = pl.pallas_call(
    _mm_body,
    grid=(XP // BLK,),
    in_specs=[
        pl.BlockSpec((BLK, D_IN), lambda i: (i, 0)),
        pl.BlockSpec((D_IN, D_OUT), lambda i: (0, 0)),
        pl.BlockSpec((2, BLK, 1), lambda i: (0, i, 0)),
    ],
    out_specs=pl.BlockSpec((BLK, D_OUT), lambda i: (i, 0)),
    out_shape=jax.ShapeDtypeStruct((XP, D_OUT), jnp.float32),
)


GPB = 10


def _fin_body(ap_ref, dg_ref, b_ref, pa_ref, pool_ref, anc_ref):
    y = ap_ref[0] + ap_ref[1]
    deg = dg_ref[0] + dg_ref[1]
    norm = lax.rsqrt(jnp.maximum(deg, 1.0))
    y = y * norm + b_ref[...]
    a = pa_ref[0, 0]
    y = jnp.where(y > 0, y, a * y)
    for g in range(GPB):
        yg = y[g * STRIDE:(g + 1) * STRIDE, :]
        pool_ref[g, 0, :] = jnp.sum(yg[:NPG - 1, :], axis=0) * (1.0 / (NPG - 1))
        anc_ref[g, 0, :] = yg[NPG - 1, :]


_final_call = pl.pallas_call(
    _fin_body,
    grid=(B // GPB,),
    in_specs=[
        pl.BlockSpec((2, GPB * STRIDE, D_OUT), lambda g: (0, g, 0)),
        pl.BlockSpec((2, GPB * STRIDE, 1), lambda g: (0, g, 0)),
        pl.BlockSpec((D_OUT,), lambda g: (0,)),
        pl.BlockSpec((1, 1), lambda g: (0, 0)),
    ],
    out_specs=[
        pl.BlockSpec((GPB, 1, D_OUT), lambda g: (g, 0, 0)),
        pl.BlockSpec((GPB, 1, D_OUT), lambda g: (g, 0, 0)),
    ],
    out_shape=[
        jax.ShapeDtypeStruct((B, 1, D_OUT), jnp.float32),
        jax.ShapeDtypeStruct((B, 1, D_OUT), jnp.float32),
    ],
)


def kernel(x, edge_index, W, b, prelu_a):
    src = edge_index[0].astype(jnp.int32)
    dst = edge_index[1].astype(jnp.int32)
    dst_r = dst + (STRIDE - NPG) * (dst // NPG)
    src_p = jnp.concatenate(
        [src, jnp.zeros((PAD,), jnp.int32)]).reshape(NT, CH, K)
    dst_p = jnp.concatenate(
        [dst_r, jnp.full((PAD,), NPG, jnp.int32)]).reshape(NT, CH, K)

    deg_kernel, agg_kernel = _sc_kernels()
    src_pd = src_p.reshape(NT, DCH, DSC)
    dst_pd = dst_p.reshape(NT, DCH, DSC)
    od, idd = deg_kernel(src_pd, dst_pd)

    xp = jnp.pad(x, ((0, XP - N), (0, 0)))
    od_col = od.reshape(NC, HL, 1)
    h = _matmul_call(xp, W, od_col)

    h3 = h.reshape(1, XP, D_OUT)
    h4 = h.reshape(XP // SC_ROWS, SC_ROWS, D_OUT)
    aggp = agg_kernel(h3, h4, src_p, dst_p)

    idg_col = idd.reshape(NC, HL, 1)
    pa = jnp.reshape(prelu_a, (1, 1)).astype(jnp.float32)
    pool, anchor = _final_call(aggp, idg_col, b, pa)
    return pool.reshape(B, D_OUT), anchor.reshape(B, D_OUT)


# speedup vs baseline: 1.0292x; 1.0292x over previous
import functools

import jax
import jax.numpy as jnp
from jax import lax
from jax.experimental import pallas as pl
from jax.experimental.pallas import tpu as pltpu
from jax.experimental.pallas import tpu_sc as plsc

N = 10000
E = 320000
D_IN = 128
D_OUT = 64
B = 100
NPG = 100
STRIDE = 104
NR = B * STRIDE
NC = 2
NSUB = 16
NT = NC * NSUB
K = 128
CH = 80
EPT = CH * K
EP = NT * EPT
PAD = EP - E
HL = 10496
HSTR = HL // NSUB
ABUF = 10496
ASTR = ABUF // NSUB
JC = 1
SC_ROWS = JC * K
NSTEP = CH // JC
AZ = 41
HROWS = 640
DSC = 512
DCH = EPT // DSC
XP = 10240
BLK = 1024

def _deg_body(src_hbm, dst_hbm, out_hbm, idx_s, idx_d, ones_v, zv, hist_o, hist_i):
    cid = lax.axis_index("c")
    sid = lax.axis_index("s")
    wid = cid * NSUB + sid
    pltpu.sync_copy(src_hbm.at[wid], idx_s)
    pltpu.sync_copy(dst_hbm.at[wid], idx_d)

    def fill_ones(i, _):
        ones_v[0, pl.ds(i * 16, 16)] = jnp.full((16,), 1.0, jnp.float32)
        return 0

    lax.fori_loop(0, DSC // 16, fill_ones, 0)

    def fill_zero(i, _):
        zv[0, pl.ds(i * 16, 16)] = jnp.zeros((16,), jnp.float32)
        return 0

    lax.fori_loop(0, HSTR // 16, fill_zero, 0)
    pltpu.sync_copy(zv, hist_o.at[pl.ds(0, 1), pl.ds(sid * HSTR, HSTR)])
    pltpu.sync_copy(zv, hist_i.at[pl.ds(0, 1), pl.ds(sid * HSTR, HSTR)])
    plsc.subcore_barrier()

    def chunk(c, _):
        pltpu.sync_copy(ones_v, hist_o.at[idx_s.at[pl.ds(c, 1)]], add=True)
        pltpu.sync_copy(ones_v, hist_i.at[idx_d.at[pl.ds(c, 1)]], add=True)
        return 0

    lax.fori_loop(0, DCH, chunk, 0)
    plsc.subcore_barrier()
    pltpu.sync_copy(hist_o.at[pl.ds(0, 1), pl.ds(sid * HSTR, HSTR)], zv)
    pltpu.sync_copy(zv, out_hbm.at[pl.ds(2 * cid, 1), pl.ds(sid * HSTR, HSTR)])
    pltpu.sync_copy(hist_i.at[pl.ds(0, 1), pl.ds(sid * HSTR, HSTR)], zv)
    pltpu.sync_copy(zv, out_hbm.at[pl.ds(2 * cid + 1, 1), pl.ds(sid * HSTR, HSTR)])


def _agg_body(h_hbm, src_hbm, dst_hbm, out_hbm,
              idx_s, idx_d, rows, zv, h_sp, agg,
              sem_ga, sem_gb, sem_sa, sem_sb):
    cid = lax.axis_index("c")
    sid = lax.axis_index("s")
    wid = cid * NSUB + sid
    pltpu.sync_copy(src_hbm.at[wid], idx_s)
    pltpu.sync_copy(dst_hbm.at[wid], idx_d)

    def fill_zero(i, _):
        zv[i // 4, pl.ds((i % 4) * 16, 16)] = jnp.zeros((16,), jnp.float32)
        return 0

    lax.fori_loop(0, AZ * 4, fill_zero, 0)
    for q in range(ASTR // AZ):
        pltpu.sync_copy(zv, agg.at[0, pl.ds(sid * ASTR + q * AZ, AZ)])
    for q in range(HROWS // SC_ROWS):
        base = sid * HROWS + q * SC_ROWS
        pltpu.sync_copy(h_hbm.at[0, pl.ds(base, SC_ROWS)], rows.at[0, 0])
        pltpu.sync_copy(rows.at[0, 0], h_sp.at[0, pl.ds(base, SC_ROWS)])
    plsc.subcore_barrier()

    g_sems = (sem_ga, sem_gb)
    s_sems = (sem_sa, sem_sb)

    def wait_buf(b, sem):
        pltpu.make_async_copy(h_hbm.at[pl.ds(0, 1), pl.ds(0, SC_ROWS)],
                              rows.at[b], sem).wait()

    def issue_gather(s, b):
        pltpu.async_copy(h_sp.at[idx_s.at[pl.ds(s, 1)]], rows.at[b],
                         g_sems[b])

    def issue_scatter(s, b):
        pltpu.async_copy(rows.at[b], agg.at[idx_d.at[pl.ds(s, 1)]],
                         s_sems[b], add=True)

    issue_gather(0, 0)
    issue_gather(1, 1)
    for s in range(NSTEP):
        b = s % 2
        wait_buf(b, g_sems[b])
        issue_scatter(s, b)
        if s + 2 < NSTEP:
            wait_buf(b, s_sems[b])
            issue_gather(s + 2, b)
    wait_buf(0, s_sems[0])
    wait_buf(1, s_sems[1])
    plsc.subcore_barrier()
    for q in range(ASTR // AZ):
        pltpu.sync_copy(agg.at[0, pl.ds(sid * ASTR + q * AZ, AZ)], zv)
        pltpu.sync_copy(zv, out_hbm.at[cid, pl.ds(sid * ASTR + q * AZ, AZ)])


@functools.cache
def _sc_kernels():
    mesh = plsc.VectorSubcoreMesh(
        core_axis_name="c", subcore_axis_name="s",
        num_cores=NC, num_subcores=NSUB,
    )
    params = pltpu.CompilerParams(use_tc_tiling_on_sc=False)
    deg_kernel = pl.kernel(
        _deg_body,
        out_type=jax.ShapeDtypeStruct((NC * 2, HL), jnp.float32),
        mesh=mesh,
        compiler_params=params,
        scratch_types=[
            pltpu.VMEM((DCH, DSC), jnp.int32),
            pltpu.VMEM((DCH, DSC), jnp.int32),
            pltpu.VMEM((1, DSC), jnp.float32),
            pltpu.VMEM((1, HSTR), jnp.float32),
            pltpu.VMEM_SHARED((1, HL), jnp.float32),
            pltpu.VMEM_SHARED((1, HL), jnp.float32),
        ],
    )
    agg_kernel = pl.kernel(
        _agg_body,
        out_type=jax.ShapeDtypeStruct((NC, ABUF, D_OUT), jnp.float32),
        mesh=mesh,
        compiler_params=params,
        scratch_types=[
            pltpu.VMEM((NSTEP, SC_ROWS), jnp.int32),
            pltpu.VMEM((NSTEP, SC_ROWS), jnp.int32),
            pltpu.VMEM((2, 1, SC_ROWS, D_OUT), jnp.float32),
            pltpu.VMEM((AZ, D_OUT), jnp.float32),
            pltpu.VMEM_SHARED((1, XP, D_OUT), jnp.float32),
            pltpu.VMEM_SHARED((1, ABUF, D_OUT), jnp.float32),
            pltpu.SemaphoreType.DMA,
            pltpu.SemaphoreType.DMA,
            pltpu.SemaphoreType.DMA,
            pltpu.SemaphoreType.DMA,
        ],
    )
    return deg_kernel, agg_kernel


def _mm_body(x_ref, w_ref, od_ref, h_ref):
    od = od_ref[0] + od_ref[1]
    pid = pl.program_id(0)
    iota = lax.broadcasted_iota(jnp.int32, (BLK, 1), 0)
    od = jnp.where((pid == 0) & (iota == 0), od - float(PAD), od)
    norm = lax.rsqrt(jnp.maximum(od, 1.0))
    h_ref[...] = jnp.dot(x_ref[...] * norm, w_ref[...],
                         preferred_element_type=jnp.float32)


_matmul_call = pl.pallas_call(
    _mm_body,
    grid=(XP // BLK,),
    in_specs=[
        pl.BlockSpec((BLK, D_IN), lambda i: (i, 0)),
        pl.BlockSpec((D_IN, D_OUT), lambda i: (0, 0)),
        pl.BlockSpec((2, BLK, 1), lambda i: (0, i, 0)),
    ],
    out_specs=pl.BlockSpec((BLK, D_OUT), lambda i: (i, 0)),
    out_shape=jax.ShapeDtypeStruct((XP, D_OUT), jnp.float32),
)


GPB = 10


def _fin_body(ap_ref, dg_ref, b_ref, pa_ref, pool_ref, anc_ref):
    y = ap_ref[0] + ap_ref[1]
    deg = dg_ref[0] + dg_ref[1]
    norm = lax.rsqrt(jnp.maximum(deg, 1.0))
    y = y * norm + b_ref[...]
    a = pa_ref[0, 0]
    y = jnp.where(y > 0, y, a * y)
    for g in range(GPB):
        yg = y[g * STRIDE:(g + 1) * STRIDE, :]
        pool_ref[g, 0, :] = jnp.sum(yg[:NPG - 1, :], axis=0) * (1.0 / (NPG - 1))
        anc_ref[g, 0, :] = yg[NPG - 1, :]


_final_call = pl.pallas_call(
    _fin_body,
    grid=(B // GPB,),
    in_specs=[
        pl.BlockSpec((2, GPB * STRIDE, D_OUT), lambda g: (0, g, 0)),
        pl.BlockSpec((2, GPB * STRIDE, 1), lambda g: (0, g, 0)),
        pl.BlockSpec((D_OUT,), lambda g: (0,)),
        pl.BlockSpec((1, 1), lambda g: (0, 0)),
    ],
    out_specs=[
        pl.BlockSpec((GPB, 1, D_OUT), lambda g: (g, 0, 0)),
        pl.BlockSpec((GPB, 1, D_OUT), lambda g: (g, 0, 0)),
    ],
    out_shape=[
        jax.ShapeDtypeStruct((B, 1, D_OUT), jnp.float32),
        jax.ShapeDtypeStruct((B, 1, D_OUT), jnp.float32),
    ],
)


def kernel(x, edge_index, W, b, prelu_a):
    src = edge_index[0].astype(jnp.int32)
    dst = edge_index[1].astype(jnp.int32)
    dst_r = dst + (STRIDE - NPG) * (dst // NPG)
    src_p = jnp.concatenate(
        [src, jnp.zeros((PAD,), jnp.int32)]).reshape(NT, CH, K)
    dst_p = jnp.concatenate(
        [dst_r, jnp.full((PAD,), NPG, jnp.int32)]).reshape(NT, CH, K)

    deg_kernel, agg_kernel = _sc_kernels()
    src_pd = src_p.reshape(NT, DCH, DSC)
    dst_pd = dst_p.reshape(NT, DCH, DSC)
    degs = deg_kernel(src_pd, dst_pd).reshape(NC, 2, HL)

    xp = jnp.pad(x, ((0, XP - N), (0, 0)))
    od_col = degs[:, 0, :XP].reshape(2, XP, 1)
    h = _matmul_call(xp, W, od_col)

    h3 = h.reshape(1, XP, D_OUT)
    aggp = agg_kernel(h3, src_p, dst_p)

    idg_col = degs[:, 1, :NR].reshape(2, NR, 1)
    pa = jnp.reshape(prelu_a, (1, 1)).astype(jnp.float32)
    pool, anchor = _final_call(aggp, idg_col, b, pa)
    return pool.reshape(B, D_OUT), anchor.reshape(B, D_OUT)


# speedup vs baseline: 1.0583x; 1.0282x over previous
import functools

import jax
import jax.numpy as jnp
from jax import lax
from jax.experimental import pallas as pl
from jax.experimental.pallas import tpu as pltpu
from jax.experimental.pallas import tpu_sc as plsc

N = 10000
E = 320000
D_IN = 128
D_OUT = 64
B = 100
NPG = 100
STRIDE = 104
NR = B * STRIDE
NC = 2
NSUB = 16
NT = NC * NSUB
K = 128
CH = 80
EPT = CH * K
EP = NT * EPT
PAD = EP - E
HL = 10496
HSTR = HL // NSUB
ABUF = 10496
ASTR = ABUF // NSUB
JC = 1
SC_ROWS = JC * K
NSTEP = CH // JC
AZ = 41
HROWS = 640
DSC = 512
DCH = EPT // DSC
XP = 10240
BLK = 2048

def _deg_body(src_hbm, dst_hbm, out_hbm, idx_s, idx_d, ones_v, zv, hist_o, hist_i):
    cid = lax.axis_index("c")
    sid = lax.axis_index("s")
    wid = cid * NSUB + sid
    pltpu.sync_copy(src_hbm.at[wid], idx_s)
    pltpu.sync_copy(dst_hbm.at[wid], idx_d)

    def fill_ones(i, _):
        ones_v[0, pl.ds(i * 16, 16)] = jnp.full((16,), 1.0, jnp.float32)
        return 0

    lax.fori_loop(0, DSC // 16, fill_ones, 0)

    def fill_zero(i, _):
        zv[0, pl.ds(i * 16, 16)] = jnp.zeros((16,), jnp.float32)
        return 0

    lax.fori_loop(0, HSTR // 16, fill_zero, 0)
    pltpu.sync_copy(zv, hist_o.at[pl.ds(0, 1), pl.ds(sid * HSTR, HSTR)])
    pltpu.sync_copy(zv, hist_i.at[pl.ds(0, 1), pl.ds(sid * HSTR, HSTR)])
    plsc.subcore_barrier()

    def chunk(c, _):
        pltpu.sync_copy(ones_v, hist_o.at[idx_s.at[pl.ds(c, 1)]], add=True)
        pltpu.sync_copy(ones_v, hist_i.at[idx_d.at[pl.ds(c, 1)]], add=True)
        return 0

    lax.fori_loop(0, DCH, chunk, 0)
    plsc.subcore_barrier()
    pltpu.sync_copy(hist_o.at[pl.ds(0, 1), pl.ds(sid * HSTR, HSTR)], zv)
    pltpu.sync_copy(zv, out_hbm.at[pl.ds(2 * cid, 1), pl.ds(sid * HSTR, HSTR)])
    pltpu.sync_copy(hist_i.at[pl.ds(0, 1), pl.ds(sid * HSTR, HSTR)], zv)
    pltpu.sync_copy(zv, out_hbm.at[pl.ds(2 * cid + 1, 1), pl.ds(sid * HSTR, HSTR)])


def _agg_body(h_hbm, src_hbm, dst_hbm, out_hbm,
              idx_s, idx_d, rows, zv, h_sp, agg,
              sem_ga, sem_gb, sem_sa, sem_sb):
    cid = lax.axis_index("c")
    sid = lax.axis_index("s")
    wid = cid * NSUB + sid
    pltpu.sync_copy(src_hbm.at[wid], idx_s)
    pltpu.sync_copy(dst_hbm.at[wid], idx_d)

    def fill_zero(i, _):
        zv[i // 4, pl.ds((i % 4) * 16, 16)] = jnp.zeros((16,), jnp.float32)
        return 0

    lax.fori_loop(0, AZ * 4, fill_zero, 0)
    for q in range(ASTR // AZ):
        pltpu.sync_copy(zv, agg.at[0, pl.ds(sid * ASTR + q * AZ, AZ)])
    for q in range(HROWS // SC_ROWS):
        base = sid * HROWS + q * SC_ROWS
        pltpu.sync_copy(h_hbm.at[0, pl.ds(base, SC_ROWS)], rows.at[0, 0])
        pltpu.sync_copy(rows.at[0, 0], h_sp.at[0, pl.ds(base, SC_ROWS)])
    plsc.subcore_barrier()

    g_sems = (sem_ga, sem_gb)
    s_sems = (sem_sa, sem_sb)

    def wait_buf(b, sem):
        pltpu.make_async_copy(h_hbm.at[pl.ds(0, 1), pl.ds(0, SC_ROWS)],
                              rows.at[b], sem).wait()

    def issue_gather(s, b):
        pltpu.async_copy(h_sp.at[idx_s.at[pl.ds(s, 1)]], rows.at[b],
                         g_sems[b])

    def issue_scatter(s, b):
        pltpu.async_copy(rows.at[b], agg.at[idx_d.at[pl.ds(s, 1)]],
                         s_sems[b], add=True)

    issue_gather(0, 0)
    issue_gather(1, 1)
    for s in range(NSTEP):
        b = s % 2
        wait_buf(b, g_sems[b])
        issue_scatter(s, b)
        if s + 2 < NSTEP:
            wait_buf(b, s_sems[b])
            issue_gather(s + 2, b)
    wait_buf(0, s_sems[0])
    wait_buf(1, s_sems[1])
    plsc.subcore_barrier()
    for q in range(ASTR // AZ):
        pltpu.sync_copy(agg.at[0, pl.ds(sid * ASTR + q * AZ, AZ)], zv)
        pltpu.sync_copy(zv, out_hbm.at[cid, pl.ds(sid * ASTR + q * AZ, AZ)])


@functools.cache
def _sc_kernels():
    mesh = plsc.VectorSubcoreMesh(
        core_axis_name="c", subcore_axis_name="s",
        num_cores=NC, num_subcores=NSUB,
    )
    params = pltpu.CompilerParams(use_tc_tiling_on_sc=False)
    deg_kernel = pl.kernel(
        _deg_body,
        out_type=jax.ShapeDtypeStruct((NC * 2, HL), jnp.float32),
        mesh=mesh,
        compiler_params=params,
        scratch_types=[
            pltpu.VMEM((DCH, DSC), jnp.int32),
            pltpu.VMEM((DCH, DSC), jnp.int32),
            pltpu.VMEM((1, DSC), jnp.float32),
            pltpu.VMEM((1, HSTR), jnp.float32),
            pltpu.VMEM_SHARED((1, HL), jnp.float32),
            pltpu.VMEM_SHARED((1, HL), jnp.float32),
        ],
    )
    agg_kernel = pl.kernel(
        _agg_body,
        out_type=jax.ShapeDtypeStruct((NC, ABUF, D_OUT), jnp.float32),
        mesh=mesh,
        compiler_params=params,
        scratch_types=[
            pltpu.VMEM((NSTEP, SC_ROWS), jnp.int32),
            pltpu.VMEM((NSTEP, SC_ROWS), jnp.int32),
            pltpu.VMEM((2, 1, SC_ROWS, D_OUT), jnp.float32),
            pltpu.VMEM((AZ, D_OUT), jnp.float32),
            pltpu.VMEM_SHARED((1, XP, D_OUT), jnp.float32),
            pltpu.VMEM_SHARED((1, ABUF, D_OUT), jnp.float32),
            pltpu.SemaphoreType.DMA,
            pltpu.SemaphoreType.DMA,
            pltpu.SemaphoreType.DMA,
            pltpu.SemaphoreType.DMA,
        ],
    )
    return deg_kernel, agg_kernel


def _mm_body(x_ref, w_ref, od_ref, h_ref):
    od = od_ref[0] + od_ref[1]
    pid = pl.program_id(0)
    iota = lax.broadcasted_iota(jnp.int32, (BLK, 1), 0)
    od = jnp.where((pid == 0) & (iota == 0), od - float(PAD), od)
    norm = lax.rsqrt(jnp.maximum(od, 1.0))
    h_ref[...] = jnp.dot(x_ref[...] * norm, w_ref[...],
                         preferred_element_type=jnp.float32)


_matmul_call = pl.pallas_call(
    _mm_body,
    grid=(XP // BLK,),
    in_specs=[
        pl.BlockSpec((BLK, D_IN), lambda i: (i, 0)),
        pl.BlockSpec((D_IN, D_OUT), lambda i: (0, 0)),
        pl.BlockSpec((2, BLK, 1), lambda i: (0, i, 0)),
    ],
    out_specs=pl.BlockSpec((BLK, D_OUT), lambda i: (i, 0)),
    out_shape=jax.ShapeDtypeStruct((XP, D_OUT), jnp.float32),
)


GPB = 25


def _fin_body(ap_ref, dg_ref, b_ref, pa_ref, pool_ref, anc_ref):
    y = ap_ref[0] + ap_ref[1]
    deg = dg_ref[0] + dg_ref[1]
    norm = lax.rsqrt(jnp.maximum(deg, 1.0))
    y = y * norm + b_ref[...]
    a = pa_ref[0, 0]
    y = jnp.where(y > 0, y, a * y)
    for g in range(GPB):
        yg = y[g * STRIDE:(g + 1) * STRIDE, :]
        pool_ref[g, 0, :] = jnp.sum(yg[:NPG - 1, :], axis=0) * (1.0 / (NPG - 1))
        anc_ref[g, 0, :] = yg[NPG - 1, :]


_final_call = pl.pallas_call(
    _fin_body,
    grid=(B // GPB,),
    in_specs=[
        pl.BlockSpec((2, GPB * STRIDE, D_OUT), lambda g: (0, g, 0)),
        pl.BlockSpec((2, GPB * STRIDE, 1), lambda g: (0, g, 0)),
        pl.BlockSpec((D_OUT,), lambda g: (0,)),
        pl.BlockSpec((1, 1), lambda g: (0, 0)),
    ],
    out_specs=[
        pl.BlockSpec((GPB, 1, D_OUT), lambda g: (g, 0, 0)),
        pl.BlockSpec((GPB, 1, D_OUT), lambda g: (g, 0, 0)),
    ],
    out_shape=[
        jax.ShapeDtypeStruct((B, 1, D_OUT), jnp.float32),
        jax.ShapeDtypeStruct((B, 1, D_OUT), jnp.float32),
    ],
)


def kernel(x, edge_index, W, b, prelu_a):
    src = edge_index[0].astype(jnp.int32)
    dst = edge_index[1].astype(jnp.int32)
    dst_r = dst + (STRIDE - NPG) * (dst // NPG)
    src_p = jnp.concatenate(
        [src, jnp.zeros((PAD,), jnp.int32)]).reshape(NT, CH, K)
    dst_p = jnp.concatenate(
        [dst_r, jnp.full((PAD,), NPG, jnp.int32)]).reshape(NT, CH, K)

    deg_kernel, agg_kernel = _sc_kernels()
    src_pd = src_p.reshape(NT, DCH, DSC)
    dst_pd = dst_p.reshape(NT, DCH, DSC)
    degs = deg_kernel(src_pd, dst_pd).reshape(NC, 2, HL)

    xp = jnp.pad(x, ((0, XP - N), (0, 0)))
    od_col = degs[:, 0, :XP].reshape(2, XP, 1)
    h = _matmul_call(xp, W, od_col)

    h3 = h.reshape(1, XP, D_OUT)
    aggp = agg_kernel(h3, src_p, dst_p)

    idg_col = degs[:, 1, :NR].reshape(2, NR, 1)
    pa = jnp.reshape(prelu_a, (1, 1)).astype(jnp.float32)
    pool, anchor = _final_call(aggp, idg_col, b, pa)
    return pool.reshape(B, D_OUT), anchor.reshape(B, D_OUT)
